# baseline (speedup 1.0000x reference)
import jax
import jax.numpy as jnp
from jax import lax
from jax.experimental import pallas as pl
from jax.experimental.pallas import tpu as pltpu
from jax.experimental.pallas import tpu_sc as plsc

B = 4096
L = 200
C = 64
PAD = 1
LANES = 16
NUM_CORES = 2
NUM_SUBCORES = 16
NW = NUM_CORES * NUM_SUBCORES
SEQ_PER_W = B // NW
LA = 128
LB = 80
OVL = LA - (L - LB)


def _body(x_hbm, table_hbm, out_hbm, len_hbm,
          xall, rows_a, rows_b, tr, len_v, sem):
  wid = lax.axis_index("s") * NUM_CORES + lax.axis_index("c")
  base = pl.multiple_of(wid * SEQ_PER_W, SEQ_PER_W)

  iota = lax.iota(jnp.int32, LANES)
  cvecs = [cb * LANES + iota for cb in range(C // LANES)]
  ovl_mask = iota >= OVL
  lane0 = iota == 0

  pltpu.sync_copy(x_hbm.at[pl.ds(base, SEQ_PER_W)], xall)

  def seq_body(i, _):
    b = base + i
    pltpu.async_copy(table_hbm.at[xall.at[i, pl.ds(0, LA)]], rows_a,
                     sem).wait()
    pltpu.async_copy(table_hbm.at[xall.at[i, pl.ds(L - LB, LB)]], rows_b,
                     sem).wait()

    acc = jnp.zeros((LANES,), jnp.int32)
    for k in range(LA // LANES):
      v = xall[i, pl.ds(k * LANES, LANES)]
      acc = acc + jnp.where(v != PAD, 1, 0)
    for k in range(LB // LANES):
      v = xall[i, pl.ds(L - LB + k * LANES, LANES)]
      cond = (v != PAD) & ovl_mask if k == 0 else (v != PAD)
      acc = acc + jnp.where(cond, 1, 0)
    cnt = jnp.sum(acc)
    plsc.store_scatter(len_v, [iota * 0 + i],
                       jnp.zeros((LANES,), jnp.int32) + cnt, mask=lane0)

    def tr_a(l, lvec):
      for cb in range(C // LANES):
        plsc.store_scatter(tr, [cvecs[cb], lvec],
                           rows_a[l, pl.ds(cb * LANES, LANES)])
      return lvec + 1

    def tr_b(l, lvec):
      for cb in range(C // LANES):
        plsc.store_scatter(tr, [cvecs[cb], lvec],
                           rows_b[l, pl.ds(cb * LANES, LANES)])
      return lvec + 1

    lvec = lax.fori_loop(0, LA, tr_a, jnp.zeros((LANES,), jnp.int32))
    lax.fori_loop(OVL, LB, tr_b, lvec)

    pltpu.sync_copy(tr, out_hbm.at[b])
    return 0

  lax.fori_loop(0, SEQ_PER_W, seq_body, 0)
  pltpu.sync_copy(len_v, len_hbm.at[pl.ds(base, SEQ_PER_W)])


@jax.jit
def _run(x, table):
  mesh = plsc.VectorSubcoreMesh(core_axis_name="c", subcore_axis_name="s")
  f = pl.kernel(
      _body,
      out_type=(
          jax.ShapeDtypeStruct((B, C, L), jnp.float32),
          jax.ShapeDtypeStruct((B,), jnp.int32),
      ),
      mesh=mesh,
      scratch_types=[
          pltpu.VMEM((SEQ_PER_W, L), jnp.int32),
          pltpu.VMEM((LA, C), jnp.float32),
          pltpu.VMEM((LB, C), jnp.float32),
          pltpu.VMEM((C, L), jnp.float32),
          pltpu.VMEM((SEQ_PER_W,), jnp.int32),
          pltpu.SemaphoreType.DMA,
      ],
      compiler_params=pltpu.CompilerParams(use_tc_tiling_on_sc=False,
                                           needs_layout_passes=False),
  )
  return f(x, table)


def kernel(x, table):
  fmap, lengths = _run(x.astype(jnp.int32), table)
  return fmap, lengths
